# speedup vs baseline: 1.3720x; 1.3720x over previous
import functools

import jax
import jax.numpy as jnp
from jax.experimental import pallas as pl
from jax.experimental.pallas import tpu as pltpu

E = 8
TOPK = 2
H = 1024
DFF = 512
NSH = 2
EE = E + NSH

BT = 512


def _moe_body(gate_w_ref, x_ref, wg_ref, wu_ref, wd_ref, y_ref):
    e = pl.program_id(1)
    x = x_ref[...]

    logits = jax.lax.dot_general(
        x, gate_w_ref[...], (((1,), (1,)), ((), ())),
        preferred_element_type=jnp.float32)
    m = jnp.max(logits, axis=-1, keepdims=True)
    p = jnp.exp(logits - m)
    scores = p / jnp.sum(p, axis=-1, keepdims=True)
    lane = jax.lax.broadcasted_iota(jnp.int32, scores.shape, 1)
    i1 = jnp.argmax(scores, axis=-1)
    s1 = jnp.max(scores, axis=-1)
    masked = jnp.where(lane == i1[:, None], -jnp.inf, scores)
    i2 = jnp.argmax(masked, axis=-1)
    s2 = jnp.max(masked, axis=-1)
    se = jnp.sum(jnp.where(lane == e, scores, 0.0), axis=-1)
    member = (i1 == e) | (i2 == e)
    w_routed = jnp.where(member, se / (s1 + s2 + 1e-20), 0.0)
    w = jnp.where(e < E, w_routed, 1.0)

    hg = jax.lax.dot_general(x, wg_ref[0], (((1,), (0,)), ((), ())),
                             preferred_element_type=jnp.float32)
    hu = jax.lax.dot_general(x, wu_ref[0], (((1,), (0,)), ((), ())),
                             preferred_element_type=jnp.float32)
    hact = (hg * jax.lax.logistic(hg)) * hu
    out = jax.lax.dot_general(hact, wd_ref[0], (((1,), (0,)), ((), ())),
                              preferred_element_type=jnp.float32)
    contrib = out * w[:, None]

    @pl.when(e == 0)
    def _init():
        y_ref[...] = contrib

    @pl.when(e != 0)
    def _acc():
        y_ref[...] += contrib


def kernel(hidden_states, gate_w, w_gate, w_up, w_down, sw_gate, sw_up, sw_down):
    b, s, h = hidden_states.shape
    x = hidden_states.reshape(-1, h)
    n = x.shape[0]

    swg = sw_gate.reshape(h, NSH, DFF).transpose(1, 0, 2)
    swu = sw_up.reshape(h, NSH, DFF).transpose(1, 0, 2)
    swd = sw_down.reshape(NSH, DFF, h)
    WG = jnp.concatenate([w_gate, swg], axis=0)
    WU = jnp.concatenate([w_up, swu], axis=0)
    WD = jnp.concatenate([w_down, swd], axis=0)

    t_blocks = n // BT
    y = pl.pallas_call(
        _moe_body,
        grid=(t_blocks, EE),
        in_specs=[
            pl.BlockSpec((E, h), lambda t, e: (0, 0)),
            pl.BlockSpec((BT, h), lambda t, e: (t, 0)),
            pl.BlockSpec((1, h, DFF), lambda t, e: (e, 0, 0)),
            pl.BlockSpec((1, h, DFF), lambda t, e: (e, 0, 0)),
            pl.BlockSpec((1, DFF, h), lambda t, e: (e, 0, 0)),
        ],
        out_specs=pl.BlockSpec((BT, h), lambda t, e: (t, 0)),
        out_shape=jax.ShapeDtypeStruct((n, h), jnp.float32),
        compiler_params=pltpu.CompilerParams(
            dimension_semantics=("parallel", "arbitrary")),
    )(gate_w, x, WG, WU, WD)
    return y.reshape(b, s, h)


# speedup vs baseline: 1.3748x; 1.0020x over previous
import functools

import jax
import jax.numpy as jnp
from jax.experimental import pallas as pl
from jax.experimental.pallas import tpu as pltpu

E = 8
TOPK = 2
H = 1024
DFF = 512
NSH = 2
EE = E + NSH

BT = 512


def _moe_body(gate_w_ref, x_ref, wg_ref, wu_ref, wd_ref, y_ref):
    e = pl.program_id(1)
    x = x_ref[...]

    logits = jax.lax.dot_general(
        x, gate_w_ref[...], (((1,), (1,)), ((), ())),
        preferred_element_type=jnp.float32)
    m = jnp.max(logits, axis=-1, keepdims=True)
    p = jnp.exp(logits - m)
    scores = p / jnp.sum(p, axis=-1, keepdims=True)
    lane = jax.lax.broadcasted_iota(jnp.int32, scores.shape, 1)
    i1 = jnp.argmax(scores, axis=-1)
    s1 = jnp.max(scores, axis=-1)
    masked = jnp.where(lane == i1[:, None], -jnp.inf, scores)
    i2 = jnp.argmax(masked, axis=-1)
    s2 = jnp.max(masked, axis=-1)
    se = jnp.sum(jnp.where(lane == e, scores, 0.0), axis=-1)
    member = (i1 == e) | (i2 == e)
    w_routed = jnp.where(member, se / (s1 + s2 + 1e-20), 0.0)
    w = jnp.where(e < E, w_routed, 1.0)

    xb = x.astype(jnp.bfloat16)
    hg = jax.lax.dot_general(xb, wg_ref[0].astype(jnp.bfloat16),
                             (((1,), (0,)), ((), ())),
                             preferred_element_type=jnp.float32)
    hu = jax.lax.dot_general(xb, wu_ref[0].astype(jnp.bfloat16),
                             (((1,), (0,)), ((), ())),
                             preferred_element_type=jnp.float32)
    hact = (hg * jax.lax.logistic(hg)) * hu
    out = jax.lax.dot_general(hact.astype(jnp.bfloat16),
                              wd_ref[0].astype(jnp.bfloat16),
                              (((1,), (0,)), ((), ())),
                              preferred_element_type=jnp.float32)
    contrib = out * w[:, None]

    @pl.when(e == 0)
    def _init():
        y_ref[...] = contrib

    @pl.when(e != 0)
    def _acc():
        y_ref[...] += contrib


def kernel(hidden_states, gate_w, w_gate, w_up, w_down, sw_gate, sw_up, sw_down):
    b, s, h = hidden_states.shape
    x = hidden_states.reshape(-1, h)
    n = x.shape[0]

    swg = sw_gate.reshape(h, NSH, DFF).transpose(1, 0, 2)
    swu = sw_up.reshape(h, NSH, DFF).transpose(1, 0, 2)
    swd = sw_down.reshape(NSH, DFF, h)
    WG = jnp.concatenate([w_gate, swg], axis=0)
    WU = jnp.concatenate([w_up, swu], axis=0)
    WD = jnp.concatenate([w_down, swd], axis=0)

    t_blocks = n // BT
    y = pl.pallas_call(
        _moe_body,
        grid=(t_blocks, EE),
        in_specs=[
            pl.BlockSpec((E, h), lambda t, e: (0, 0)),
            pl.BlockSpec((BT, h), lambda t, e: (t, 0)),
            pl.BlockSpec((1, h, DFF), lambda t, e: (e, 0, 0)),
            pl.BlockSpec((1, h, DFF), lambda t, e: (e, 0, 0)),
            pl.BlockSpec((1, DFF, h), lambda t, e: (e, 0, 0)),
        ],
        out_specs=pl.BlockSpec((BT, h), lambda t, e: (t, 0)),
        out_shape=jax.ShapeDtypeStruct((n, h), jnp.float32),
        compiler_params=pltpu.CompilerParams(
            dimension_semantics=("parallel", "arbitrary")),
    )(gate_w, x, WG, WU, WD)
    return y.reshape(b, s, h)


# speedup vs baseline: 1.4485x; 1.0536x over previous
import functools

import jax
import jax.numpy as jnp
from jax.experimental import pallas as pl
from jax.experimental.pallas import tpu as pltpu

E = 8
TOPK = 2
H = 1024
DFF = 512
NSH = 2
EE = E + NSH

BT = 1024


def _moe_body(gate_w_ref, x_ref, wg_ref, wu_ref, wd_ref, y_ref):
    e = pl.program_id(1)
    x = x_ref[...]

    logits = jax.lax.dot_general(
        x, gate_w_ref[...], (((1,), (1,)), ((), ())),
        preferred_element_type=jnp.float32)
    m = jnp.max(logits, axis=-1, keepdims=True)
    p = jnp.exp(logits - m)
    scores = p / jnp.sum(p, axis=-1, keepdims=True)
    lane = jax.lax.broadcasted_iota(jnp.int32, scores.shape, 1)
    i1 = jnp.argmax(scores, axis=-1)
    s1 = jnp.max(scores, axis=-1)
    masked = jnp.where(lane == i1[:, None], -jnp.inf, scores)
    i2 = jnp.argmax(masked, axis=-1)
    s2 = jnp.max(masked, axis=-1)
    se = jnp.sum(jnp.where(lane == e, scores, 0.0), axis=-1)
    member = (i1 == e) | (i2 == e)
    w_routed = jnp.where(member, se / (s1 + s2 + 1e-20), 0.0)
    w = jnp.where(e < E, w_routed, 1.0)

    xb = x.astype(jnp.bfloat16)
    hg = jax.lax.dot_general(xb, wg_ref[0].astype(jnp.bfloat16),
                             (((1,), (0,)), ((), ())),
                             preferred_element_type=jnp.float32)
    hu = jax.lax.dot_general(xb, wu_ref[0].astype(jnp.bfloat16),
                             (((1,), (0,)), ((), ())),
                             preferred_element_type=jnp.float32)
    hact = (hg * jax.lax.logistic(hg)) * hu
    out = jax.lax.dot_general(hact.astype(jnp.bfloat16),
                              wd_ref[0].astype(jnp.bfloat16),
                              (((1,), (0,)), ((), ())),
                              preferred_element_type=jnp.float32)
    contrib = out * w[:, None]

    @pl.when(e == 0)
    def _init():
        y_ref[...] = contrib

    @pl.when(e != 0)
    def _acc():
        y_ref[...] += contrib


def kernel(hidden_states, gate_w, w_gate, w_up, w_down, sw_gate, sw_up, sw_down):
    b, s, h = hidden_states.shape
    x = hidden_states.reshape(-1, h)
    n = x.shape[0]

    swg = sw_gate.reshape(h, NSH, DFF).transpose(1, 0, 2)
    swu = sw_up.reshape(h, NSH, DFF).transpose(1, 0, 2)
    swd = sw_down.reshape(NSH, DFF, h)
    WG = jnp.concatenate([w_gate, swg], axis=0)
    WU = jnp.concatenate([w_up, swu], axis=0)
    WD = jnp.concatenate([w_down, swd], axis=0)

    t_blocks = n // BT
    y = pl.pallas_call(
        _moe_body,
        grid=(t_blocks, EE),
        in_specs=[
            pl.BlockSpec((E, h), lambda t, e: (0, 0)),
            pl.BlockSpec((BT, h), lambda t, e: (t, 0)),
            pl.BlockSpec((1, h, DFF), lambda t, e: (e, 0, 0)),
            pl.BlockSpec((1, h, DFF), lambda t, e: (e, 0, 0)),
            pl.BlockSpec((1, DFF, h), lambda t, e: (e, 0, 0)),
        ],
        out_specs=pl.BlockSpec((BT, h), lambda t, e: (t, 0)),
        out_shape=jax.ShapeDtypeStruct((n, h), jnp.float32),
        compiler_params=pltpu.CompilerParams(
            dimension_semantics=("parallel", "arbitrary")),
    )(gate_w, x, WG, WU, WD)
    return y.reshape(b, s, h)
